# baseline (speedup 1.0000x reference)
import functools

import jax
import jax.numpy as jnp
from jax import lax
from jax.experimental import pallas as pl
from jax.experimental.pallas import tpu as pltpu
from jax.experimental.pallas import tpu_sc as plsc

B, S, D = 1, 2048, 1024
H = 16
DH = D // H
E = 8
DFF = 4096
K = (B * S * 2) // E

SBLK = 256
QBLK = 512
NQ = S // QBLK
FBLK = 1024
NF = DFF // FBLK

BF = jnp.bfloat16
F32 = jnp.float32


def _qkv_body(x_ref, g_ref, b_ref, wq_ref, wk_ref, wv_ref, bq_ref, bk_ref,
              bv_ref, q_ref, k_ref, v_ref):
    x = x_ref[...]
    m = jnp.mean(x, axis=-1, keepdims=True)
    var = jnp.mean((x - m) ** 2, axis=-1, keepdims=True)
    xn = ((x - m) * jax.lax.rsqrt(var + 1e-5) * g_ref[...] + b_ref[...]).astype(BF)
    q = (jnp.dot(xn, wq_ref[...], preferred_element_type=F32)
         + bq_ref[...]).astype(BF)
    k = (jnp.dot(xn, wk_ref[...], preferred_element_type=F32)
         + bk_ref[...]).astype(BF)
    v = (jnp.dot(xn, wv_ref[...], preferred_element_type=F32)
         + bv_ref[...]).astype(BF)
    q_ref[...] = q.reshape(SBLK, H, DH).transpose(1, 0, 2)
    k_ref[...] = k.reshape(SBLK, H, DH).transpose(1, 0, 2)
    v_ref[...] = v.reshape(SBLK, H, DH).transpose(1, 0, 2)


def _qkv(x, g, b, wq, wk, wv, bq, bk, bv):
    n = S // SBLK
    row = pl.BlockSpec((SBLK, D), lambda i: (i, 0))
    full = pl.BlockSpec((D, D), lambda i: (0, 0))
    vec = pl.BlockSpec((1, D), lambda i: (0, 0))
    return pl.pallas_call(
        _qkv_body,
        grid=(n,),
        in_specs=[row, vec, vec, full, full, full, vec, vec, vec],
        out_specs=[pl.BlockSpec((H, SBLK, DH), lambda i: (0, i, 0))] * 3,
        out_shape=[jax.ShapeDtypeStruct((H, S, DH), BF)] * 3,
    )(x, g, b, wq, wk, wv, bq, bk, bv)


def _make_attn_body(qb, klen):
    def body(q_ref, k_ref, v_ref, o_ref):
        q = q_ref[0]
        kk = k_ref[0]
        s = jax.lax.dot_general(q, kk, (((1,), (1,)), ((), ())),
                                preferred_element_type=F32) * 0.125
        row = jax.lax.broadcasted_iota(jnp.int32, (QBLK, klen), 0) + qb * QBLK
        col = jax.lax.broadcasted_iota(jnp.int32, (QBLK, klen), 1)
        s = jnp.where(row >= col, s, -1e9)
        m = jnp.max(s, axis=-1, keepdims=True)
        e = jnp.exp(s - m)
        p = (e / jnp.sum(e, axis=-1, keepdims=True)).astype(BF)
        o_ref[0] = jnp.dot(p, v_ref[0], preferred_element_type=F32).astype(BF)
    return body


def _attention(q3, k3, v3):
    outs = []
    for qb in range(NQ):
        klen = (qb + 1) * QBLK
        o = pl.pallas_call(
            _make_attn_body(qb, klen),
            grid=(H,),
            in_specs=[
                pl.BlockSpec((1, QBLK, DH), lambda h, qb=qb: (h, qb, 0)),
                pl.BlockSpec((1, klen, DH), lambda h: (h, 0, 0)),
                pl.BlockSpec((1, klen, DH), lambda h: (h, 0, 0)),
            ],
            out_specs=pl.BlockSpec((1, QBLK, DH), lambda h: (h, 0, 0)),
            out_shape=jax.ShapeDtypeStruct((H, QBLK, DH), BF),
        )(q3, k3, v3)
        outs.append(o)
    return jnp.concatenate(outs, axis=1)


def _proj_body(a_ref, wo_ref, bo_ref, x_ref, g_ref, b_ref, wg_ref,
               h_ref, h2_ref, sc_ref):
    a = a_ref[...].transpose(1, 0, 2).reshape(SBLK, D)
    h = x_ref[...] + jnp.dot(a, wo_ref[...],
                             preferred_element_type=F32) + bo_ref[...]
    h_ref[...] = h
    m = jnp.mean(h, axis=-1, keepdims=True)
    var = jnp.mean((h - m) ** 2, axis=-1, keepdims=True)
    h2 = (h - m) * jax.lax.rsqrt(var + 1e-5) * g_ref[...] + b_ref[...]
    h2_ref[...] = h2
    lg = jnp.dot(h2, wg_ref[...], preferred_element_type=F32)
    lg = lg - jnp.max(lg, axis=-1, keepdims=True)
    el = jnp.exp(lg)
    sc_ref[...] = el / jnp.sum(el, axis=-1, keepdims=True)


def _proj(attn, wo, bo, x, g, b, wg):
    n = S // SBLK
    row = pl.BlockSpec((SBLK, D), lambda i: (i, 0))
    return pl.pallas_call(
        _proj_body,
        grid=(n,),
        in_specs=[pl.BlockSpec((H, SBLK, DH), lambda i: (0, i, 0)),
                  pl.BlockSpec((D, D), lambda i: (0, 0)),
                  pl.BlockSpec((1, D), lambda i: (0, 0)),
                  row,
                  pl.BlockSpec((1, D), lambda i: (0, 0)),
                  pl.BlockSpec((1, D), lambda i: (0, 0)),
                  pl.BlockSpec((D, E), lambda i: (0, 0))],
        out_specs=[row, row, pl.BlockSpec((SBLK, E), lambda i: (i, 0))],
        out_shape=[jax.ShapeDtypeStruct((S, D), F32),
                   jax.ShapeDtypeStruct((S, D), F32),
                   jax.ShapeDtypeStruct((S, E), F32)],
    )(attn, wo, bo, x, g, b, wg)


def _topk_body(sc_ref, sel_ref, rank_ref, sct_ref):
    bits = jax.lax.bitcast_convert_type(sc_ref[...], jnp.int32)

    def step(i, kth):
        cand = kth | jax.lax.shift_left(1, 30 - i)
        cnt = jnp.sum((bits >= cand).astype(F32), axis=0, keepdims=True)
        return jnp.where(cnt >= K, cand, kth)

    kth = jax.lax.fori_loop(0, 31, step, jnp.zeros((1, E), jnp.int32))

    gt = bits > kth
    eq = bits == kth
    tril = (jax.lax.broadcasted_iota(jnp.int32, (S, S), 0)
            >= jax.lax.broadcasted_iota(jnp.int32, (S, S), 1)).astype(F32)
    need = K - jnp.sum(gt.astype(F32), axis=0, keepdims=True)
    cumeq = jnp.dot(tril, eq.astype(F32), preferred_element_type=F32)
    sel = jnp.logical_or(gt, jnp.logical_and(eq, cumeq <= need)).astype(F32)
    cumsel = jnp.dot(tril, sel, preferred_element_type=F32)
    sel_ref[...] = sel.T
    rank_ref[...] = cumsel.T - 1.0
    sct_ref[...] = sc_ref[...].T


def _topk(sc):
    return pl.pallas_call(
        _topk_body,
        grid=(1,),
        in_specs=[pl.BlockSpec((S, E), lambda i: (0, 0))],
        out_specs=[pl.BlockSpec((E, S), lambda i: (0, 0))] * 3,
        out_shape=[jax.ShapeDtypeStruct((E, S), F32)] * 3,
    )(sc)


def _compact_body(selt_ref, rankt_ref, sct_ref, idx_ref, gate_ref):
    iota0 = jax.lax.broadcasted_iota(jnp.int32, (K, S), 0)
    tokrow = jax.lax.broadcasted_iota(jnp.int32, (K, S), 1)
    for e in range(E):
        rank_b = jnp.broadcast_to(rankt_ref[pl.ds(e, 1)], (K, S))
        sel_b = jnp.broadcast_to(selt_ref[pl.ds(e, 1)], (K, S))
        sc_b = jnp.broadcast_to(sct_ref[pl.ds(e, 1)], (K, S))
        ot = jnp.logical_and(iota0 == rank_b.astype(jnp.int32), sel_b > 0.5)
        idx_ref[e] = jnp.sum(jnp.where(ot, tokrow, 0), axis=1)
        gate_ref[e] = jnp.sum(jnp.where(ot, sc_b, 0.0), axis=1)


def _compact(selt, rankt, sct):
    full = pl.BlockSpec((E, S), lambda: (0, 0))
    return pl.pallas_call(
        _compact_body,
        in_specs=[full, full, full],
        out_specs=[pl.BlockSpec((E, K), lambda: (0, 0))] * 2,
        out_shape=[jax.ShapeDtypeStruct((E, K), jnp.int32),
                   jax.ShapeDtypeStruct((E, K), F32)],
    )(selt, rankt, sct)


_SC_MESH = plsc.VectorSubcoreMesh(core_axis_name="c", subcore_axis_name="s")


@functools.partial(
    pl.kernel, mesh=_SC_MESH,
    out_type=jax.ShapeDtypeStruct((E * K, D), F32),
    scratch_types=[pltpu.VMEM((128,), jnp.int32), pltpu.VMEM((64, D), F32),
                   pltpu.SemaphoreType.DMA],
)
def _sc_gather(idxf, h2, xs, idx_v, rows_v, sem):
    wid = lax.axis_index("s") * 2 + lax.axis_index("c")
    base = wid * 128
    pltpu.sync_copy(idxf.at[pl.ds(base, 128)], idx_v)
    for j in range(2):
        pltpu.async_copy(h2.at[idx_v.at[pl.ds(j * 64, 64)]], rows_v, sem).wait()
        pltpu.sync_copy(rows_v, xs.at[pl.ds(base + j * 64, 64)])


def _scatter_body(ys_ref, selc_ref, rankc_ref, h_ref, y_ref):
    @pl.when(pl.program_id(0) == 0)
    def _():
        y_ref[...] = h_ref[...]

    iota_kr = jax.lax.broadcasted_iota(jnp.int32, (S, K), 1).astype(F32)
    pt = jnp.where(rankc_ref[0] == iota_kr,
                   jnp.broadcast_to(selc_ref[0], (S, K)), 0.0).astype(BF)
    y_ref[...] += jnp.dot(pt, ys_ref[0].astype(BF), preferred_element_type=F32)


def _scatter_call(ys3, selc, rankc, h):
    colv = pl.BlockSpec((1, S, 1), lambda e: (e, 0, 0))
    return pl.pallas_call(
        _scatter_body,
        grid=(E,),
        in_specs=[pl.BlockSpec((1, K, D), lambda e: (e, 0, 0)),
                  colv, colv,
                  pl.BlockSpec((S, D), lambda e: (0, 0))],
        out_specs=pl.BlockSpec((S, D), lambda e: (0, 0)),
        out_shape=jax.ShapeDtypeStruct((S, D), F32),
    )(ys3, selc, rankc, h)


def _moe_body(xs_ref, g_ref, w1_ref, b1_ref, w2_ref, b2_ref, ys_ref,
              xsb_ref, acc_ref):
    f = pl.program_id(1)

    @pl.when(f == 0)
    def _():
        xsb_ref[...] = xs_ref[0].astype(BF)
        acc_ref[...] = jnp.broadcast_to(b2_ref[0], (K, D))

    part = jnp.maximum(
        jnp.dot(xsb_ref[...], w1_ref[0], preferred_element_type=F32)
        + b1_ref[0], 0.0).astype(BF)
    acc_ref[...] += jnp.dot(part, w2_ref[0], preferred_element_type=F32)

    @pl.when(f == NF - 1)
    def _():
        ys_ref[0] = acc_ref[...] * g_ref[0]


def _moe_call(xs3, g3, w1, b1, w2, b2):
    return pl.pallas_call(
        _moe_body,
        grid=(E, NF),
        in_specs=[pl.BlockSpec((1, K, D), lambda e, f: (e, 0, 0)),
                  pl.BlockSpec((1, K, 1), lambda e, f: (e, 0, 0)),
                  pl.BlockSpec((1, D, FBLK), lambda e, f: (e, 0, f)),
                  pl.BlockSpec((1, 1, FBLK), lambda e, f: (e, 0, f)),
                  pl.BlockSpec((1, FBLK, D), lambda e, f: (e, f, 0)),
                  pl.BlockSpec((1, 1, D), lambda e, f: (e, 0, 0))],
        out_specs=pl.BlockSpec((1, K, D), lambda e, f: (e, 0, 0)),
        out_shape=jax.ShapeDtypeStruct((E, K, D), F32),
        scratch_shapes=[pltpu.VMEM((K, D), BF),
                        pltpu.VMEM((K, D), F32)],
    )(xs3, g3, w1, b1, w2, b2)


def kernel(x, ln1_g, ln1_b, ln2_g, ln2_b, Wq, bq, Wk, bk, Wv, bv, Wo, bo,
           Wg, W1, b1, W2, b2):
    x2 = x.reshape(S, D)
    q3, k3, v3 = _qkv(x2, ln1_g.reshape(1, D), ln1_b.reshape(1, D),
                      Wq.astype(BF), Wk.astype(BF), Wv.astype(BF),
                      bq.reshape(1, D), bk.reshape(1, D), bv.reshape(1, D))
    o3 = _attention(q3, k3, v3)
    h, h2, sc = _proj(o3, Wo.astype(BF), bo.reshape(1, D), x2,
                      ln2_g.reshape(1, D), ln2_b.reshape(1, D), Wg)
    selt, rankt, sct = _topk(sc)
    idxg, gateg = _compact(selt, rankt, sct)
    xs = _sc_gather(idxg.reshape(E * K), h2)
    ys = _moe_call(xs.reshape(E, K, D), gateg.reshape(E, K, 1),
                   W1.astype(BF), b1.reshape(E, 1, DFF),
                   W2.astype(BF), b2.reshape(E, 1, D))
    out = _scatter_call(ys, selt.reshape(E, S, 1), rankt.reshape(E, S, 1), h)
    return out.reshape(B, S, D)


# speedup vs baseline: 1.0769x; 1.0769x over previous
import functools

import jax
import jax.numpy as jnp
from jax import lax
from jax.experimental import pallas as pl
from jax.experimental.pallas import tpu as pltpu
from jax.experimental.pallas import tpu_sc as plsc

B, S, D = 1, 2048, 1024
H = 16
DH = D // H
E = 8
DFF = 4096
K = (B * S * 2) // E

SBLK = 256
QBLK = 512
NQ = S // QBLK
FBLK = 1024
NF = DFF // FBLK

BF = jnp.bfloat16
F32 = jnp.float32


def _qkv_body(x_ref, g_ref, b_ref, wq_ref, wk_ref, wv_ref, bq_ref, bk_ref,
              bv_ref, q_ref, k_ref, v_ref):
    x = x_ref[...]
    m = jnp.mean(x, axis=-1, keepdims=True)
    var = jnp.mean((x - m) ** 2, axis=-1, keepdims=True)
    xn = ((x - m) * jax.lax.rsqrt(var + 1e-5) * g_ref[...] + b_ref[...]).astype(BF)
    q = (jnp.dot(xn, wq_ref[...], preferred_element_type=F32)
         + bq_ref[...]).astype(BF)
    k = (jnp.dot(xn, wk_ref[...], preferred_element_type=F32)
         + bk_ref[...]).astype(BF)
    v = (jnp.dot(xn, wv_ref[...], preferred_element_type=F32)
         + bv_ref[...]).astype(BF)
    for h in range(H):
        q_ref[h] = q[:, h * DH:(h + 1) * DH]
        k_ref[h] = k[:, h * DH:(h + 1) * DH]
        v_ref[h] = v[:, h * DH:(h + 1) * DH]


def _qkv(x, g, b, wq, wk, wv, bq, bk, bv):
    n = S // SBLK
    row = pl.BlockSpec((SBLK, D), lambda i: (i, 0))
    full = pl.BlockSpec((D, D), lambda i: (0, 0))
    vec = pl.BlockSpec((1, D), lambda i: (0, 0))
    return pl.pallas_call(
        _qkv_body,
        grid=(n,),
        in_specs=[row, vec, vec, full, full, full, vec, vec, vec],
        out_specs=[pl.BlockSpec((H, SBLK, DH), lambda i: (0, i, 0))] * 3,
        out_shape=[jax.ShapeDtypeStruct((H, S, DH), BF)] * 3,
    )(x, g, b, wq, wk, wv, bq, bk, bv)


def _make_attn_body(qb, klen):
    def body(q_ref, k_ref, v_ref, o_ref):
        q = q_ref[0]
        kk = k_ref[0]
        s = jax.lax.dot_general(q, kk, (((1,), (1,)), ((), ())),
                                preferred_element_type=F32) * 0.125
        row = jax.lax.broadcasted_iota(jnp.int32, (QBLK, klen), 0) + qb * QBLK
        col = jax.lax.broadcasted_iota(jnp.int32, (QBLK, klen), 1)
        s = jnp.where(row >= col, s, -1e9)
        m = jnp.max(s, axis=-1, keepdims=True)
        e = jnp.exp(s - m)
        p = (e / jnp.sum(e, axis=-1, keepdims=True)).astype(BF)
        o_ref[0] = jnp.dot(p, v_ref[0], preferred_element_type=F32).astype(BF)
    return body


def _attention(q3, k3, v3):
    outs = []
    for qb in range(NQ):
        klen = (qb + 1) * QBLK
        o = pl.pallas_call(
            _make_attn_body(qb, klen),
            grid=(H,),
            in_specs=[
                pl.BlockSpec((1, QBLK, DH), lambda h, qb=qb: (h, qb, 0)),
                pl.BlockSpec((1, klen, DH), lambda h: (h, 0, 0)),
                pl.BlockSpec((1, klen, DH), lambda h: (h, 0, 0)),
            ],
            out_specs=pl.BlockSpec((1, QBLK, DH), lambda h: (h, 0, 0)),
            out_shape=jax.ShapeDtypeStruct((H, QBLK, DH), BF),
        )(q3, k3, v3)
        outs.append(o)
    return jnp.concatenate(outs, axis=1)


def _proj_body(a_ref, wo_ref, bo_ref, x_ref, g_ref, b_ref, wg_ref,
               h_ref, h2_ref, sc_ref):
    a = jnp.concatenate([a_ref[h] for h in range(H)], axis=1)
    h = x_ref[...] + jnp.dot(a, wo_ref[...],
                             preferred_element_type=F32) + bo_ref[...]
    h_ref[...] = h
    m = jnp.mean(h, axis=-1, keepdims=True)
    var = jnp.mean((h - m) ** 2, axis=-1, keepdims=True)
    h2 = (h - m) * jax.lax.rsqrt(var + 1e-5) * g_ref[...] + b_ref[...]
    h2_ref[...] = h2
    lg = jnp.dot(h2, wg_ref[...], preferred_element_type=F32)
    lg = lg - jnp.max(lg, axis=-1, keepdims=True)
    el = jnp.exp(lg)
    sc_ref[...] = el / jnp.sum(el, axis=-1, keepdims=True)


def _proj(attn, wo, bo, x, g, b, wg):
    n = S // SBLK
    row = pl.BlockSpec((SBLK, D), lambda i: (i, 0))
    return pl.pallas_call(
        _proj_body,
        grid=(n,),
        in_specs=[pl.BlockSpec((H, SBLK, DH), lambda i: (0, i, 0)),
                  pl.BlockSpec((D, D), lambda i: (0, 0)),
                  pl.BlockSpec((1, D), lambda i: (0, 0)),
                  row,
                  pl.BlockSpec((1, D), lambda i: (0, 0)),
                  pl.BlockSpec((1, D), lambda i: (0, 0)),
                  pl.BlockSpec((D, E), lambda i: (0, 0))],
        out_specs=[row, row, pl.BlockSpec((SBLK, E), lambda i: (i, 0))],
        out_shape=[jax.ShapeDtypeStruct((S, D), F32),
                   jax.ShapeDtypeStruct((S, D), F32),
                   jax.ShapeDtypeStruct((S, E), F32)],
    )(attn, wo, bo, x, g, b, wg)


def _topk_body(sc_ref, sel_ref, rank_ref, sct_ref):
    bits = jax.lax.bitcast_convert_type(sc_ref[...], jnp.int32)

    def step(i, kth):
        cand = kth | jax.lax.shift_left(1, 30 - i)
        cnt = jnp.sum((bits >= cand).astype(F32), axis=0, keepdims=True)
        return jnp.where(cnt >= K, cand, kth)

    kth = jax.lax.fori_loop(0, 31, step, jnp.zeros((1, E), jnp.int32))

    gt = bits > kth
    eq = bits == kth
    tril = (jax.lax.broadcasted_iota(jnp.int32, (S, S), 0)
            >= jax.lax.broadcasted_iota(jnp.int32, (S, S), 1)).astype(F32)
    need = K - jnp.sum(gt.astype(F32), axis=0, keepdims=True)
    cumeq = jnp.dot(tril, eq.astype(F32), preferred_element_type=F32)
    sel = jnp.logical_or(gt, jnp.logical_and(eq, cumeq <= need)).astype(F32)
    cumsel = jnp.dot(tril, sel, preferred_element_type=F32)
    sel_ref[...] = sel.T
    rank_ref[...] = cumsel.T - 1.0
    sct_ref[...] = sc_ref[...].T


def _topk(sc):
    return pl.pallas_call(
        _topk_body,
        grid=(1,),
        in_specs=[pl.BlockSpec((S, E), lambda i: (0, 0))],
        out_specs=[pl.BlockSpec((E, S), lambda i: (0, 0))] * 3,
        out_shape=[jax.ShapeDtypeStruct((E, S), F32)] * 3,
    )(sc)


def _compact_body(selt_ref, rankt_ref, sct_ref, idx_ref, gate_ref):
    iota0 = jax.lax.broadcasted_iota(jnp.int32, (K, S), 0)
    tokrow = jax.lax.broadcasted_iota(jnp.int32, (K, S), 1)
    for e in range(E):
        rank_b = jnp.broadcast_to(rankt_ref[pl.ds(e, 1)], (K, S))
        sel_b = jnp.broadcast_to(selt_ref[pl.ds(e, 1)], (K, S))
        sc_b = jnp.broadcast_to(sct_ref[pl.ds(e, 1)], (K, S))
        ot = jnp.logical_and(iota0 == rank_b.astype(jnp.int32), sel_b > 0.5)
        idx_ref[e] = jnp.sum(jnp.where(ot, tokrow, 0), axis=1)
        gate_ref[e] = jnp.sum(jnp.where(ot, sc_b, 0.0), axis=1)


def _compact(selt, rankt, sct):
    full = pl.BlockSpec((E, S), lambda: (0, 0))
    return pl.pallas_call(
        _compact_body,
        in_specs=[full, full, full],
        out_specs=[pl.BlockSpec((E, K), lambda: (0, 0))] * 2,
        out_shape=[jax.ShapeDtypeStruct((E, K), jnp.int32),
                   jax.ShapeDtypeStruct((E, K), F32)],
    )(selt, rankt, sct)


_SC_MESH = plsc.VectorSubcoreMesh(core_axis_name="c", subcore_axis_name="s")


@functools.partial(
    pl.kernel, mesh=_SC_MESH,
    out_type=jax.ShapeDtypeStruct((E * K, D), F32),
    scratch_types=[pltpu.VMEM((128,), jnp.int32), pltpu.VMEM((64, D), F32),
                   pltpu.SemaphoreType.DMA],
)
def _sc_gather(idxf, h2, xs, idx_v, rows_v, sem):
    wid = lax.axis_index("s") * 2 + lax.axis_index("c")
    base = wid * 128
    pltpu.sync_copy(idxf.at[pl.ds(base, 128)], idx_v)
    for j in range(2):
        pltpu.async_copy(h2.at[idx_v.at[pl.ds(j * 64, 64)]], rows_v, sem).wait()
        pltpu.sync_copy(rows_v, xs.at[pl.ds(base + j * 64, 64)])


def _moe_body(xs_ref, g_ref, selc_ref, rankc_ref, h_ref, w1_ref, b1_ref,
              w2_ref, b2_ref, y_ref, xsb_ref, acc_ref):
    e = pl.program_id(0)
    f = pl.program_id(1)

    @pl.when(jnp.logical_and(e == 0, f == 0))
    def _():
        y_ref[...] = h_ref[...]

    @pl.when(f == 0)
    def _():
        xsb_ref[...] = xs_ref[0].astype(BF)
        acc_ref[...] = jnp.broadcast_to(b2_ref[0], (K, D))

    part = jnp.maximum(
        jnp.dot(xsb_ref[...], w1_ref[0], preferred_element_type=F32)
        + b1_ref[0], 0.0).astype(BF)
    acc_ref[...] += jnp.dot(part, w2_ref[0], preferred_element_type=F32)

    @pl.when(f == NF - 1)
    def _():
        ys = (acc_ref[...] * g_ref[0]).astype(BF)
        iota_kr = jax.lax.broadcasted_iota(jnp.int32, (S, K), 1)
        pt = jnp.where(rankc_ref[0].astype(jnp.int32) == iota_kr,
                       jnp.broadcast_to(selc_ref[0], (S, K)), 0.0).astype(BF)
        y_ref[...] += jnp.dot(pt, ys, preferred_element_type=F32)


def _moe_call(xs3, g3, selc, rankc, h, w1, b1, w2, b2):
    colv = pl.BlockSpec((1, S, 1), lambda e, f: (e, 0, 0))
    return pl.pallas_call(
        _moe_body,
        grid=(E, NF),
        in_specs=[pl.BlockSpec((1, K, D), lambda e, f: (e, 0, 0)),
                  pl.BlockSpec((1, K, 1), lambda e, f: (e, 0, 0)),
                  colv, colv,
                  pl.BlockSpec((S, D), lambda e, f: (0, 0)),
                  pl.BlockSpec((1, D, FBLK), lambda e, f: (e, 0, f)),
                  pl.BlockSpec((1, 1, FBLK), lambda e, f: (e, 0, f)),
                  pl.BlockSpec((1, FBLK, D), lambda e, f: (e, f, 0)),
                  pl.BlockSpec((1, 1, D), lambda e, f: (e, 0, 0))],
        out_specs=pl.BlockSpec((S, D), lambda e, f: (0, 0)),
        out_shape=jax.ShapeDtypeStruct((S, D), F32),
        scratch_shapes=[pltpu.VMEM((K, D), BF),
                        pltpu.VMEM((K, D), F32)],
    )(xs3, g3, selc, rankc, h, w1, b1, w2, b2)


def kernel(x, ln1_g, ln1_b, ln2_g, ln2_b, Wq, bq, Wk, bk, Wv, bv, Wo, bo,
           Wg, W1, b1, W2, b2):
    x2 = x.reshape(S, D)
    q3, k3, v3 = _qkv(x2, ln1_g.reshape(1, D), ln1_b.reshape(1, D),
                      Wq.astype(BF), Wk.astype(BF), Wv.astype(BF),
                      bq.reshape(1, D), bk.reshape(1, D), bv.reshape(1, D))
    o3 = _attention(q3, k3, v3)
    h, h2, sc = _proj(o3, Wo.astype(BF), bo.reshape(1, D), x2,
                      ln2_g.reshape(1, D), ln2_b.reshape(1, D), Wg)
    selt, rankt, sct = _topk(sc)
    idxg, gateg = _compact(selt, rankt, sct)
    xs = _sc_gather(idxg.reshape(E * K), h2)
    out = _moe_call(xs.reshape(E, K, D), gateg.reshape(E, K, 1),
                    selt.reshape(E, S, 1), rankt.reshape(E, S, 1), h,
                    W1.astype(BF), b1.reshape(E, 1, DFF),
                    W2.astype(BF), b2.reshape(E, 1, D))
    return out.reshape(B, S, D)


# speedup vs baseline: 1.0891x; 1.0113x over previous
import functools

import jax
import jax.numpy as jnp
from jax import lax
from jax.experimental import pallas as pl
from jax.experimental.pallas import tpu as pltpu
from jax.experimental.pallas import tpu_sc as plsc

B, S, D = 1, 2048, 1024
H = 16
DH = D // H
E = 8
DFF = 4096
K = (B * S * 2) // E

SBLK = 256
QBLK = 512
NQ = S // QBLK
FBLK = 1024
NF = DFF // FBLK

BF = jnp.bfloat16
F32 = jnp.float32


def _qkv_body(x_ref, g_ref, b_ref, wq_ref, wk_ref, wv_ref, bq_ref, bk_ref,
              bv_ref, q_ref, k_ref, v_ref):
    x = x_ref[...]
    m = jnp.mean(x, axis=-1, keepdims=True)
    var = jnp.mean((x - m) ** 2, axis=-1, keepdims=True)
    xn = ((x - m) * jax.lax.rsqrt(var + 1e-5) * g_ref[...] + b_ref[...]).astype(BF)
    q = (jnp.dot(xn, wq_ref[...], preferred_element_type=F32)
         + bq_ref[...]).astype(BF)
    k = (jnp.dot(xn, wk_ref[...], preferred_element_type=F32)
         + bk_ref[...]).astype(BF)
    v = (jnp.dot(xn, wv_ref[...], preferred_element_type=F32)
         + bv_ref[...]).astype(BF)
    for h in range(H):
        q_ref[h] = q[:, h * DH:(h + 1) * DH]
        k_ref[h] = k[:, h * DH:(h + 1) * DH]
        v_ref[h] = v[:, h * DH:(h + 1) * DH]


def _qkv(x, g, b, wq, wk, wv, bq, bk, bv):
    n = S // SBLK
    row = pl.BlockSpec((SBLK, D), lambda i: (i, 0))
    full = pl.BlockSpec((D, D), lambda i: (0, 0))
    vec = pl.BlockSpec((1, D), lambda i: (0, 0))
    return pl.pallas_call(
        _qkv_body,
        grid=(n,),
        in_specs=[row, vec, vec, full, full, full, vec, vec, vec],
        out_specs=[pl.BlockSpec((H, SBLK, DH), lambda i: (0, i, 0))] * 3,
        out_shape=[jax.ShapeDtypeStruct((H, S, DH), BF)] * 3,
    )(x, g, b, wq, wk, wv, bq, bk, bv)


def _make_attn_body(qb, klen):
    def body(q_ref, k_ref, v_ref, o_ref):
        q = q_ref[0]
        kk = k_ref[0]
        s = jax.lax.dot_general(q, kk, (((1,), (1,)), ((), ())),
                                preferred_element_type=F32) * 0.125
        row = jax.lax.broadcasted_iota(jnp.int32, (QBLK, klen), 0) + qb * QBLK
        col = jax.lax.broadcasted_iota(jnp.int32, (QBLK, klen), 1)
        s = jnp.where(row >= col, s, -1e9)
        m = jnp.max(s, axis=-1, keepdims=True)
        e = jnp.exp(s - m)
        p = (e / jnp.sum(e, axis=-1, keepdims=True)).astype(BF)
        o_ref[0] = jnp.dot(p, v_ref[0], preferred_element_type=F32).astype(BF)
    return body


def _attention(q3, k3, v3):
    outs = []
    for qb in range(NQ):
        klen = (qb + 1) * QBLK
        o = pl.pallas_call(
            _make_attn_body(qb, klen),
            grid=(H,),
            in_specs=[
                pl.BlockSpec((1, QBLK, DH), lambda h, qb=qb: (h, qb, 0)),
                pl.BlockSpec((1, klen, DH), lambda h: (h, 0, 0)),
                pl.BlockSpec((1, klen, DH), lambda h: (h, 0, 0)),
            ],
            out_specs=pl.BlockSpec((1, QBLK, DH), lambda h: (h, 0, 0)),
            out_shape=jax.ShapeDtypeStruct((H, QBLK, DH), BF),
        )(q3, k3, v3)
        outs.append(o)
    return jnp.concatenate(outs, axis=1)


def _proj_body(a_ref, wo_ref, bo_ref, x_ref, g_ref, b_ref, wg_ref,
               h_ref, h2_ref, sc_ref):
    a = jnp.concatenate([a_ref[h] for h in range(H)], axis=1)
    h = x_ref[...] + jnp.dot(a, wo_ref[...],
                             preferred_element_type=F32) + bo_ref[...]
    h_ref[...] = h
    m = jnp.mean(h, axis=-1, keepdims=True)
    var = jnp.mean((h - m) ** 2, axis=-1, keepdims=True)
    h2 = (h - m) * jax.lax.rsqrt(var + 1e-5) * g_ref[...] + b_ref[...]
    h2_ref[...] = h2
    lg = jnp.dot(h2, wg_ref[...], preferred_element_type=F32)
    lg = lg - jnp.max(lg, axis=-1, keepdims=True)
    el = jnp.exp(lg)
    sc_ref[...] = el / jnp.sum(el, axis=-1, keepdims=True)


def _proj(attn, wo, bo, x, g, b, wg):
    n = S // SBLK
    row = pl.BlockSpec((SBLK, D), lambda i: (i, 0))
    return pl.pallas_call(
        _proj_body,
        grid=(n,),
        in_specs=[pl.BlockSpec((H, SBLK, DH), lambda i: (0, i, 0)),
                  pl.BlockSpec((D, D), lambda i: (0, 0)),
                  pl.BlockSpec((1, D), lambda i: (0, 0)),
                  row,
                  pl.BlockSpec((1, D), lambda i: (0, 0)),
                  pl.BlockSpec((1, D), lambda i: (0, 0)),
                  pl.BlockSpec((D, E), lambda i: (0, 0))],
        out_specs=[row, row, pl.BlockSpec((SBLK, E), lambda i: (i, 0))],
        out_shape=[jax.ShapeDtypeStruct((S, D), F32),
                   jax.ShapeDtypeStruct((S, D), F32),
                   jax.ShapeDtypeStruct((S, E), F32)],
    )(attn, wo, bo, x, g, b, wg)


def _topk_body(sc_ref, sel_ref, rank_ref, idx_ref, gate_ref):
    bits = jax.lax.bitcast_convert_type(sc_ref[...], jnp.int32)

    def step(i, kth):
        cand = kth | jax.lax.shift_left(1, 30 - i)
        cnt = jnp.sum((bits >= cand).astype(F32), axis=0, keepdims=True)
        return jnp.where(cnt >= K, cand, kth)

    kth = jax.lax.fori_loop(0, 31, step, jnp.zeros((1, E), jnp.int32))

    gt = bits > kth
    eq = bits == kth
    tril = (jax.lax.broadcasted_iota(jnp.int32, (S, S), 0)
            >= jax.lax.broadcasted_iota(jnp.int32, (S, S), 1)).astype(F32)
    need = K - jnp.sum(gt.astype(F32), axis=0, keepdims=True)
    cumeq = jnp.dot(tril, eq.astype(F32), preferred_element_type=F32)
    sel = jnp.logical_or(gt, jnp.logical_and(eq, cumeq <= need)).astype(F32)
    cumsel = jnp.dot(tril, sel, preferred_element_type=F32)
    selT = sel.T
    rankT = cumsel.T - 1.0
    scT = sc_ref[...].T
    sel_ref[...] = selT
    rank_ref[...] = rankT

    iota0 = jax.lax.broadcasted_iota(jnp.int32, (K, S), 0)
    tokrow = jax.lax.broadcasted_iota(jnp.int32, (K, S), 1)
    for e in range(E):
        rank_b = jnp.broadcast_to(jax.lax.slice(rankT, (e, 0), (e + 1, S)),
                                  (K, S))
        sel_b = jnp.broadcast_to(jax.lax.slice(selT, (e, 0), (e + 1, S)),
                                 (K, S))
        sc_b = jnp.broadcast_to(jax.lax.slice(scT, (e, 0), (e + 1, S)),
                                (K, S))
        ot = jnp.logical_and(iota0 == rank_b.astype(jnp.int32), sel_b > 0.5)
        idx_ref[e] = jnp.sum(jnp.where(ot, tokrow, 0), axis=1)
        gate_ref[e] = jnp.sum(jnp.where(ot, sc_b, 0.0), axis=1)


def _topk(sc):
    return pl.pallas_call(
        _topk_body,
        grid=(1,),
        in_specs=[pl.BlockSpec((S, E), lambda i: (0, 0))],
        out_specs=[pl.BlockSpec((E, S), lambda i: (0, 0))] * 2
        + [pl.BlockSpec((E, K), lambda i: (0, 0))] * 2,
        out_shape=[jax.ShapeDtypeStruct((E, S), F32)] * 2
        + [jax.ShapeDtypeStruct((E, K), jnp.int32),
           jax.ShapeDtypeStruct((E, K), F32)],
    )(sc)


_SC_MESH = plsc.VectorSubcoreMesh(core_axis_name="c", subcore_axis_name="s")


@functools.partial(
    pl.kernel, mesh=_SC_MESH,
    out_type=jax.ShapeDtypeStruct((E * K, D), F32),
    scratch_types=[pltpu.VMEM((128,), jnp.int32), pltpu.VMEM((64, D), F32),
                   pltpu.SemaphoreType.DMA],
)
def _sc_gather(idxf, h2, xs, idx_v, rows_v, sem):
    wid = lax.axis_index("s") * 2 + lax.axis_index("c")
    base = wid * 128
    pltpu.sync_copy(idxf.at[pl.ds(base, 128)], idx_v)
    for j in range(2):
        pltpu.async_copy(h2.at[idx_v.at[pl.ds(j * 64, 64)]], rows_v, sem).wait()
        pltpu.sync_copy(rows_v, xs.at[pl.ds(base + j * 64, 64)])


def _moe_body(xs_ref, g_ref, selc_ref, rankc_ref, h_ref, w1_ref, b1_ref,
              w2_ref, b2_ref, y_ref, xsb_ref, acc_ref):
    e = pl.program_id(0)
    f = pl.program_id(1)

    @pl.when(jnp.logical_and(e == 0, f == 0))
    def _():
        y_ref[...] = h_ref[...]

    @pl.when(f == 0)
    def _():
        xsb_ref[...] = xs_ref[0].astype(BF)
        acc_ref[...] = jnp.broadcast_to(b2_ref[0], (K, D))

    part = jnp.maximum(
        jnp.dot(xsb_ref[...], w1_ref[0], preferred_element_type=F32)
        + b1_ref[0], 0.0).astype(BF)
    acc_ref[...] += jnp.dot(part, w2_ref[0], preferred_element_type=F32)

    @pl.when(f == NF - 1)
    def _():
        ys = (acc_ref[...] * g_ref[0]).astype(BF)
        iota_kr = jax.lax.broadcasted_iota(jnp.int32, (S, K), 1)
        pt = jnp.where(rankc_ref[0].astype(jnp.int32) == iota_kr,
                       jnp.broadcast_to(selc_ref[0], (S, K)), 0.0).astype(BF)
        y_ref[...] += jnp.dot(pt, ys, preferred_element_type=F32)


def _moe_call(xs3, g3, selc, rankc, h, w1, b1, w2, b2):
    colv = pl.BlockSpec((1, S, 1), lambda e, f: (e, 0, 0))
    return pl.pallas_call(
        _moe_body,
        grid=(E, NF),
        in_specs=[pl.BlockSpec((1, K, D), lambda e, f: (e, 0, 0)),
                  pl.BlockSpec((1, K, 1), lambda e, f: (e, 0, 0)),
                  colv, colv,
                  pl.BlockSpec((S, D), lambda e, f: (0, 0)),
                  pl.BlockSpec((1, D, FBLK), lambda e, f: (e, 0, f)),
                  pl.BlockSpec((1, 1, FBLK), lambda e, f: (e, 0, f)),
                  pl.BlockSpec((1, FBLK, D), lambda e, f: (e, f, 0)),
                  pl.BlockSpec((1, 1, D), lambda e, f: (e, 0, 0))],
        out_specs=pl.BlockSpec((S, D), lambda e, f: (0, 0)),
        out_shape=jax.ShapeDtypeStruct((S, D), F32),
        scratch_shapes=[pltpu.VMEM((K, D), BF),
                        pltpu.VMEM((K, D), F32)],
    )(xs3, g3, selc, rankc, h, w1, b1, w2, b2)


def kernel(x, ln1_g, ln1_b, ln2_g, ln2_b, Wq, bq, Wk, bk, Wv, bv, Wo, bo,
           Wg, W1, b1, W2, b2):
    x2 = x.reshape(S, D)
    q3, k3, v3 = _qkv(x2, ln1_g.reshape(1, D), ln1_b.reshape(1, D),
                      Wq.astype(BF), Wk.astype(BF), Wv.astype(BF),
                      bq.reshape(1, D), bk.reshape(1, D), bv.reshape(1, D))
    o3 = _attention(q3, k3, v3)
    h, h2, sc = _proj(o3, Wo.astype(BF), bo.reshape(1, D), x2,
                      ln2_g.reshape(1, D), ln2_b.reshape(1, D), Wg)
    selt, rankt, idxg, gateg = _topk(sc)
    xs = _sc_gather(idxg.reshape(E * K), h2)
    out = _moe_call(xs.reshape(E, K, D), gateg.reshape(E, K, 1),
                    selt.reshape(E, S, 1), rankt.reshape(E, S, 1), h,
                    W1.astype(BF), b1.reshape(E, 1, DFF),
                    W2.astype(BF), b2.reshape(E, 1, D))
    return out.reshape(B, S, D)
